# baseline (speedup 1.0000x reference)
import functools

import jax
import jax.numpy as jnp
from jax import lax
from jax.experimental import pallas as pl
from jax.experimental.pallas import tpu as pltpu
from jax.experimental.pallas import tpu_sc as plsc

NUM_CORES = 2
NUM_SUBCORES = 16
NW = NUM_CORES * NUM_SUBCORES
BATCH = 16384
EMB = 64
ROWPAD = 128

SC_ROWS = BATCH // 2
BPW = SC_ROWS // NW
LANES = 16
NGROUP = BPW // LANES

TC_ROWS = BATCH - SC_ROWS
CH = 1024
NSTEP = TC_ROWS // CH


def _sc_body(user_emb, item_emb, user_ids, pos_item_ids, neg_item_ids,
             pos_out, neg_out,
             uid_v, pid_v, nid_v, u_v, p_v, n_v, po_v, no_v, dummy_v, sem):
    wid = lax.axis_index("s") * NUM_CORES + lax.axis_index("c")
    base = wid * BPW

    pltpu.sync_copy(user_ids.at[pl.ds(base, BPW)], uid_v)
    pltpu.sync_copy(pos_item_ids.at[pl.ds(base, BPW)], pid_v)
    pltpu.sync_copy(neg_item_ids.at[pl.ds(base, BPW)], nid_v)

    lanes = lax.iota(jnp.int32, LANES)

    def fetch_step(g, carry):
        uu16 = uid_v[pl.ds(g * LANES, LANES)]
        pp16 = pid_v[pl.ds(g * LANES, LANES)]
        nn16 = nid_v[pl.ds(g * LANES, LANES)]
        for j in range(LANES):
            r = g * LANES + j
            pltpu.async_copy(user_emb.at[uu16[j]],
                             u_v.at[r, pl.ds(0, EMB)], sem)
            pltpu.async_copy(item_emb.at[pp16[j]],
                             p_v.at[r, pl.ds(0, EMB)], sem)
            pltpu.async_copy(item_emb.at[nn16[j]],
                             n_v.at[r, pl.ds(0, EMB)], sem)
        return carry

    lax.fori_loop(0, NGROUP, fetch_step, 0)

    for _ in range(6):
        pltpu.make_async_copy(pos_out, dummy_v, sem).wait()

    def group_step(g, carry):
        rows = g * LANES + lanes
        acc_p = jnp.zeros((LANES,), jnp.float32)
        acc_n = jnp.zeros((LANES,), jnp.float32)
        for d in range(EMB):
            cols = jnp.full((LANES,), d, jnp.int32)
            uu = plsc.load_gather(u_v, [rows, cols])
            pp = plsc.load_gather(p_v, [rows, cols])
            nn = plsc.load_gather(n_v, [rows, cols])
            acc_p = acc_p + uu * pp
            acc_n = acc_n + uu * nn
        po_v[pl.ds(g * LANES, LANES)] = acc_p
        no_v[pl.ds(g * LANES, LANES)] = acc_n
        return carry

    lax.fori_loop(0, NGROUP, group_step, 0)

    pltpu.sync_copy(po_v, pos_out.at[pl.ds(base, BPW)])
    pltpu.sync_copy(no_v, neg_out.at[pl.ds(base, BPW)])


def _sc_half(user_emb, item_emb, user_ids, pos_item_ids, neg_item_ids):
    mesh = plsc.VectorSubcoreMesh(core_axis_name="c", subcore_axis_name="s")
    run = functools.partial(
        pl.kernel,
        out_type=(
            jax.ShapeDtypeStruct((SC_ROWS,), jnp.float32),
            jax.ShapeDtypeStruct((SC_ROWS,), jnp.float32),
        ),
        mesh=mesh,
        scratch_types=[
            pltpu.VMEM((BPW,), jnp.int32),
            pltpu.VMEM((BPW,), jnp.int32),
            pltpu.VMEM((BPW,), jnp.int32),
            pltpu.VMEM((BPW, ROWPAD), jnp.float32),
            pltpu.VMEM((BPW, ROWPAD), jnp.float32),
            pltpu.VMEM((BPW, ROWPAD), jnp.float32),
            pltpu.VMEM((BPW,), jnp.float32),
            pltpu.VMEM((BPW,), jnp.float32),
            pltpu.VMEM((SC_ROWS,), jnp.float32),
            pltpu.SemaphoreType.DMA,
        ],
        compiler_params=pltpu.CompilerParams(needs_layout_passes=False),
    )(_sc_body)
    return run(user_emb, item_emb, user_ids, pos_item_ids, neg_item_ids)


def _tc_body(uid_s, pid_s, nid_s, utab, itab, po, no, u_b, p_b, n_b, sem):
    def issue(r, carry):
        pltpu.async_copy(utab.at[pl.ds(uid_s[r], 1), :],
                         u_b.at[pl.ds(r, 1), :], sem)
        pltpu.async_copy(itab.at[pl.ds(pid_s[r], 1), :],
                         p_b.at[pl.ds(r, 1), :], sem)
        pltpu.async_copy(itab.at[pl.ds(nid_s[r], 1), :],
                         n_b.at[pl.ds(r, 1), :], sem)
        return carry

    lax.fori_loop(0, CH, issue, 0)
    pltpu.make_async_copy(utab.at[pl.ds(0, CH), :], u_b, sem).wait()
    pltpu.make_async_copy(itab.at[pl.ds(0, CH), :], p_b, sem).wait()
    pltpu.make_async_copy(itab.at[pl.ds(0, CH), :], n_b, sem).wait()
    u = u_b[...]
    po[...] = jnp.sum(u * p_b[...], axis=1)
    no[...] = jnp.sum(u * n_b[...], axis=1)


def _tc_half(user_emb, item_emb, user_ids, pos_item_ids, neg_item_ids):
    return pl.pallas_call(
        _tc_body,
        grid=(NSTEP,),
        in_specs=[
            pl.BlockSpec((CH,), lambda i: (i,), memory_space=pltpu.SMEM),
            pl.BlockSpec((CH,), lambda i: (i,), memory_space=pltpu.SMEM),
            pl.BlockSpec((CH,), lambda i: (i,), memory_space=pltpu.SMEM),
            pl.BlockSpec(memory_space=pl.ANY),
            pl.BlockSpec(memory_space=pl.ANY),
        ],
        out_specs=[
            pl.BlockSpec((CH,), lambda i: (i,)),
            pl.BlockSpec((CH,), lambda i: (i,)),
        ],
        out_shape=[
            jax.ShapeDtypeStruct((TC_ROWS,), jnp.float32),
            jax.ShapeDtypeStruct((TC_ROWS,), jnp.float32),
        ],
        scratch_shapes=[
            pltpu.VMEM((CH, EMB), jnp.float32),
            pltpu.VMEM((CH, EMB), jnp.float32),
            pltpu.VMEM((CH, EMB), jnp.float32),
            pltpu.SemaphoreType.DMA,
        ],
    )(user_ids, pos_item_ids, neg_item_ids, user_emb, item_emb)


@jax.jit
def _bpr(user_emb, item_emb, user_ids, pos_item_ids, neg_item_ids):
    po_sc, no_sc = _sc_half(user_emb, item_emb,
                            user_ids[:SC_ROWS],
                            pos_item_ids[:SC_ROWS],
                            neg_item_ids[:SC_ROWS])
    po_tc, no_tc = _tc_half(user_emb, item_emb,
                            user_ids[SC_ROWS:],
                            pos_item_ids[SC_ROWS:],
                            neg_item_ids[SC_ROWS:])
    return (jnp.concatenate([po_sc, po_tc]),
            jnp.concatenate([no_sc, no_tc]))


def kernel(user_emb, item_emb, user_ids, pos_item_ids, neg_item_ids):
    return _bpr(user_emb, item_emb,
                user_ids.astype(jnp.int32),
                pos_item_ids.astype(jnp.int32),
                neg_item_ids.astype(jnp.int32))


# speedup vs baseline: 1.0712x; 1.0712x over previous
import functools

import jax
import jax.numpy as jnp
from jax import lax
from jax.experimental import pallas as pl
from jax.experimental.pallas import tpu as pltpu
from jax.experimental.pallas import tpu_sc as plsc

NUM_CORES = 2
NUM_SUBCORES = 16
NW = NUM_CORES * NUM_SUBCORES
BATCH = 16384
EMB = 64
ROWPAD = 128
BPW = BATCH // NW
HALF = BPW // 2
LANES = 16
NGROUP = HALF // LANES


def _bpr_body(user_emb, item_emb, user_ids, pos_item_ids, neg_item_ids,
              pos_out, neg_out,
              uid_v, pid_v, nid_v, u_v, p_v, n_v, po_v, no_v, dummy_v, sem):
    wid = lax.axis_index("s") * NUM_CORES + lax.axis_index("c")
    base = wid * BPW

    pltpu.sync_copy(user_ids.at[pl.ds(base, BPW)], uid_v)
    pltpu.sync_copy(pos_item_ids.at[pl.ds(base, BPW)], pid_v)
    pltpu.sync_copy(neg_item_ids.at[pl.ds(base, BPW)], nid_v)

    lanes = lax.iota(jnp.int32, LANES)

    for h in range(2):
        hoff = h * HALF

        def fetch_step(g, carry):
            uu16 = uid_v[pl.ds(hoff + g * LANES, LANES)]
            pp16 = pid_v[pl.ds(hoff + g * LANES, LANES)]
            nn16 = nid_v[pl.ds(hoff + g * LANES, LANES)]
            for j in range(LANES):
                r = g * LANES + j
                pltpu.async_copy(user_emb.at[uu16[j]],
                                 u_v.at[r, pl.ds(0, EMB)], sem)
                pltpu.async_copy(item_emb.at[pp16[j]],
                                 p_v.at[r, pl.ds(0, EMB)], sem)
                pltpu.async_copy(item_emb.at[nn16[j]],
                                 n_v.at[r, pl.ds(0, EMB)], sem)
            return carry

        lax.fori_loop(0, NGROUP, fetch_step, 0)

        for _ in range(6):
            pltpu.make_async_copy(pos_out.at[pl.ds(0, HALF * EMB // 2)],
                                  dummy_v, sem).wait()

        def group_step(g, carry):
            rows = g * LANES + lanes
            acc_p = jnp.zeros((LANES,), jnp.float32)
            acc_n = jnp.zeros((LANES,), jnp.float32)
            for d in range(EMB):
                cols = jnp.full((LANES,), d, jnp.int32)
                uu = plsc.load_gather(u_v, [rows, cols])
                pp = plsc.load_gather(p_v, [rows, cols])
                nn = plsc.load_gather(n_v, [rows, cols])
                acc_p = acc_p + uu * pp
                acc_n = acc_n + uu * nn
            po_v[pl.ds(hoff + g * LANES, LANES)] = acc_p
            no_v[pl.ds(hoff + g * LANES, LANES)] = acc_n
            return carry

        lax.fori_loop(0, NGROUP, group_step, 0)

    pltpu.sync_copy(po_v, pos_out.at[pl.ds(base, BPW)])
    pltpu.sync_copy(no_v, neg_out.at[pl.ds(base, BPW)])


@jax.jit
def _bpr(user_emb, item_emb, user_ids, pos_item_ids, neg_item_ids):
    mesh = plsc.VectorSubcoreMesh(core_axis_name="c", subcore_axis_name="s")
    run = functools.partial(
        pl.kernel,
        out_type=(
            jax.ShapeDtypeStruct((BATCH,), jnp.float32),
            jax.ShapeDtypeStruct((BATCH,), jnp.float32),
        ),
        mesh=mesh,
        scratch_types=[
            pltpu.VMEM((BPW,), jnp.int32),
            pltpu.VMEM((BPW,), jnp.int32),
            pltpu.VMEM((BPW,), jnp.int32),
            pltpu.VMEM((HALF, ROWPAD), jnp.float32),
            pltpu.VMEM((HALF, ROWPAD), jnp.float32),
            pltpu.VMEM((HALF, ROWPAD), jnp.float32),
            pltpu.VMEM((BPW,), jnp.float32),
            pltpu.VMEM((BPW,), jnp.float32),
            pltpu.VMEM((HALF * EMB // 2,), jnp.float32),
            pltpu.SemaphoreType.DMA,
        ],
        compiler_params=pltpu.CompilerParams(needs_layout_passes=False),
    )(_bpr_body)
    return run(user_emb, item_emb, user_ids, pos_item_ids, neg_item_ids)


def kernel(user_emb, item_emb, user_ids, pos_item_ids, neg_item_ids):
    return _bpr(user_emb, item_emb,
                user_ids.astype(jnp.int32),
                pos_item_ids.astype(jnp.int32),
                neg_item_ids.astype(jnp.int32))
